# baseline (speedup 1.0000x reference)
import functools
import numpy as np
import jax
import jax.numpy as jnp
from jax import lax
from jax.experimental import pallas as pl
from jax.experimental.pallas import tpu as pltpu
from jax.experimental.pallas import tpu_sc as plsc

_N = 10000
_E = 160000
_MUL = 8
_NB = 10
_H = 100
_LAYERS = 3
_MSGC = 176
_MSGP = 192
_OUTC = 128
_INV_SQRT_NNORM = 0.25

_GW = 128
_EPAD = 163840
_NROWS = _EPAD // 128
_NWORK = 32
_RPW = _NROWS // _NWORK
_NBIN = 10016
_RSUB = _NBIN // 16
_WSUB = _N // 16

_BE = 1280
_BN = 1000

_PREC = lax.Precision.HIGHEST


def _selectors():
    A = np.zeros((80, _MSGP), np.float32)
    for i in range(48):
        A[i, i] = 1.0
    for k in range(16):
        for j in range(3):
            A[48 + k, 48 + 3 * k + j] = 1.0
        for j in range(5):
            A[64 + k, 96 + 5 * k + j] = 1.0
    Q = np.zeros((9, _MSGP), np.float32)
    Q[0, :48] = 1.0
    for k in range(16):
        for j in range(3):
            Q[1 + j, 48 + 3 * k + j] = 1.0
        for j in range(5):
            Q[4 + j, 96 + 5 * k + j] = 1.0
    P3 = np.zeros((32, 48), np.float32)
    P5 = np.zeros((32, 80), np.float32)
    for k in range(16):
        for j in range(3):
            P3[k, 3 * k + j] = 1.0
        for j in range(5):
            P5[16 + k, 5 * k + j] = 1.0
    cent = np.linspace(0.0, 1.0, _NB).reshape(1, _NB).astype(np.float32)
    return A, Q, P3, P5, cent


def _sc_gather(feat, idx2d):
    cin = feat.shape[1]
    n_steps = idx2d.shape[1] // _GW
    mesh = plsc.VectorSubcoreMesh(core_axis_name="c", subcore_axis_name="s")

    def body(feat_hbm, i_hbm, o_hbm):
        def inner(i_vmem, o_vmem):
            pltpu.sync_copy(feat_hbm.at[i_vmem.at[0]], o_vmem)

        pltpu.emit_pipeline(
            inner,
            grid=(n_steps,),
            in_specs=[pl.BlockSpec((1, _GW), lambda i: (0, i))],
            out_specs=[pl.BlockSpec((_GW, cin), lambda i: (i, 0))],
            core_axis_name=("c", "s"),
            dimension_semantics=(pltpu.PARALLEL,),
        )(i_hbm, o_hbm)

    f = pl.kernel(
        body,
        out_type=jax.ShapeDtypeStruct((idx2d.shape[1], cin), jnp.float32),
        mesh=mesh,
        compiler_params=pltpu.CompilerParams(use_tc_tiling_on_sc=False),
    )
    return f(feat, idx2d)


def _sc_scatter(msg, dst3):
    C = msg.shape[1]
    CW = C // 2
    RPS = _NROWS // 16

    mesh = plsc.VectorSubcoreMesh(core_axis_name="c", subcore_axis_name="s")

    def body(msg_hbm, dst_hbm, out_hbm, idx_v, blk_v, acc_sh):
        c = lax.axis_index("c")
        s = lax.axis_index("s")
        coff = c * CW

        @pl.loop(0, 128)
        def _(r):
            @pl.loop(0, CW // 16)
            def _(j):
                blk_v[r, pl.ds(j * 16, 16)] = jnp.zeros((16,), jnp.float32)

        zbase = s * _RSUB

        @pl.loop(0, 4)
        def _(k):
            pltpu.sync_copy(blk_v, acc_sh.at[pl.ds(zbase + k * 128, 128)])

        pltpu.sync_copy(
            blk_v.at[pl.ds(0, _RSUB - 512)],
            acc_sh.at[pl.ds(zbase + 512, _RSUB - 512)],
        )
        plsc.subcore_barrier()

        @pl.loop(0, RPS)
        def _(k):
            r = s * RPS + k
            pltpu.sync_copy(dst_hbm.at[r], idx_v)
            pltpu.sync_copy(
                msg_hbm.at[pl.ds(r * 128, 128), pl.ds(coff, CW)], blk_v)
            pltpu.sync_copy(blk_v, acc_sh.at[idx_v], add=True)

        plsc.subcore_barrier()

        wbase = s * _WSUB
        pltpu.sync_copy(
            acc_sh.at[pl.ds(wbase, _WSUB)],
            out_hbm.at[pl.ds(wbase, _WSUB), pl.ds(coff, CW)],
        )

    f = pl.kernel(
        body,
        out_type=jax.ShapeDtypeStruct((_N, C), jnp.float32),
        mesh=mesh,
        scratch_types=[
            pltpu.VMEM((128,), jnp.int32),
            pltpu.VMEM((128, CW), jnp.float32),
            pltpu.VMEM_SHARED((_NBIN, CW), jnp.float32),
        ],
        compiler_params=pltpu.CompilerParams(use_tc_tiling_on_sc=False),
    )
    return f(msg, dst3)


def _edge_body(ea_ref, g_ref, w1, b1, w2, b2, w3, b3, wmix, a_ref, q_ref,
               cent_ref, msg_ref):
    ea = ea_ref[...]
    r2 = jnp.sum(ea * ea, axis=1, keepdims=True) + 1e-12
    r = jnp.sqrt(r2)
    inv = 1.0 / r
    xx = ea[:, 0:1] * inv
    yy = ea[:, 1:2] * inv
    zz = ea[:, 2:3] * inv

    d = (r - cent_ref[...]) * np.float32(_NB)
    B = jnp.exp(-0.5 * d * d)

    h1 = jnp.dot(B, w1[...], precision=_PREC) + b1[...]
    h1 = h1 * jax.nn.sigmoid(h1)
    h2 = jnp.dot(h1, w2[...], precision=_PREC) + b2[...]
    h2 = h2 * jax.nn.sigmoid(h2)
    R = jnp.dot(h2, w3[...], precision=_PREC) + b3[...]

    s = g_ref[...] * R
    h = jnp.dot(s, wmix[...], precision=_PREC)
    hA = jnp.dot(h, a_ref[...], precision=_PREC)

    q = q_ref[...]
    sq3 = np.float32(np.sqrt(3.0))
    ymul = (q[0:1, :]
            + xx * q[1:2, :] + yy * q[2:3, :] + zz * q[3:4, :]
            + (sq3 * xx * yy) * q[4:5, :]
            + (sq3 * yy * zz) * q[5:6, :]
            + ((3.0 * zz * zz - 1.0) * 0.5) * q[6:7, :]
            + (sq3 * xx * zz) * q[7:8, :]
            + (sq3 * 0.5 * (xx * xx - yy * yy)) * q[8:9, :])

    msg_ref[...] = hA * ymul * np.float32(_INV_SQRT_NNORM)


def _tc_edge_msgs(eap, g, p, tag, A, Q, cent):
    cin = g.shape[1]
    grid = (_EPAD // _BE,)
    full = lambda shape: pl.BlockSpec(shape, lambda i: (0, 0))
    return pl.pallas_call(
        _edge_body,
        grid=grid,
        in_specs=[
            pl.BlockSpec((_BE, 3), lambda i: (i, 0)),
            pl.BlockSpec((_BE, cin), lambda i: (i, 0)),
            full((_NB, _H)),
            full((1, _H)),
            full((_H, _H)),
            full((1, _H)),
            full((_H, cin)),
            full((1, cin)),
            full((cin, 80)),
            full((80, _MSGP)),
            full((9, _MSGP)),
            full((1, _NB)),
        ],
        out_specs=pl.BlockSpec((_BE, _MSGP), lambda i: (i, 0)),
        out_shape=jax.ShapeDtypeStruct((_EPAD, _MSGP), jnp.float32),
    )(eap, g,
      p['W1_' + tag], p['b1_' + tag].reshape(1, _H),
      p['W2_' + tag], p['b2_' + tag].reshape(1, _H),
      p['W3_' + tag], p['b3_' + tag].reshape(1, cin),
      p['Wmix_' + tag], A, Q, cent)


def _final_body(ea_ref, g_ref, w1, b1, w2, b2, w3, b3, wout, cent_ref,
                msg_ref):
    ea = ea_ref[...]
    r2 = jnp.sum(ea * ea, axis=1, keepdims=True) + 1e-12
    r = jnp.sqrt(r2)
    d = (r - cent_ref[...]) * np.float32(_NB)
    B = jnp.exp(-0.5 * d * d)
    h1 = jnp.dot(B, w1[...], precision=_PREC) + b1[...]
    h1 = h1 * jax.nn.sigmoid(h1)
    h2 = jnp.dot(h1, w2[...], precision=_PREC) + b2[...]
    h2 = h2 * jax.nn.sigmoid(h2)
    R = jnp.dot(h2, w3[...], precision=_PREC) + b3[...]
    s = g_ref[...] * R
    msg_ref[...] = jnp.dot(s, wout[...], precision=_PREC) * np.float32(
        _INV_SQRT_NNORM)


def _tc_final_msgs(eap, g, p, cent):
    cin = g.shape[1]
    grid = (_EPAD // _BE,)
    full = lambda shape: pl.BlockSpec(shape, lambda i: (0, 0))
    return pl.pallas_call(
        _final_body,
        grid=grid,
        in_specs=[
            pl.BlockSpec((_BE, 3), lambda i: (i, 0)),
            pl.BlockSpec((_BE, cin), lambda i: (i, 0)),
            full((_NB, _H)),
            full((1, _H)),
            full((_H, _H)),
            full((1, _H)),
            full((_H, cin)),
            full((1, cin)),
            full((cin, _OUTC)),
            full((1, _NB)),
        ],
        out_specs=pl.BlockSpec((_BE, _OUTC), lambda i: (i, 0)),
        out_shape=jax.ShapeDtypeStruct((_EPAD, _OUTC), jnp.float32),
    )(eap, g,
      p['W1_f'], p['b1_f'].reshape(1, _H),
      p['W2_f'], p['b2_f'].reshape(1, _H),
      p['W3_f'], p['b3_f'].reshape(1, cin),
      p['Wout'], cent)


def _act_body(o_ref, p3_ref, p5_ref, feat_ref):
    o = o_ref[...]
    se = o[:, :8]
    se = se * jax.nn.sigmoid(se)
    so = jnp.tanh(o[:, 8:16])
    g = jax.nn.sigmoid(o[:, 16:48])
    v1 = o[:, 48:96] * jnp.dot(g, p3_ref[...], precision=_PREC)
    v2 = o[:, 96:176] * jnp.dot(g, p5_ref[...], precision=_PREC)
    feat_ref[...] = jnp.concatenate([se, so, v1, v2], axis=1)


def _tc_act(o, P3, P5):
    grid = (_N // _BN,)
    full = lambda shape: pl.BlockSpec(shape, lambda i: (0, 0))
    return pl.pallas_call(
        _act_body,
        grid=grid,
        in_specs=[
            pl.BlockSpec((_BN, _MSGP), lambda i: (i, 0)),
            full((32, 48)),
            full((32, 80)),
        ],
        out_specs=pl.BlockSpec((_BN, 144), lambda i: (i, 0)),
        out_shape=jax.ShapeDtypeStruct((_N, 144), jnp.float32),
    )(o, P3, P5)


def kernel(x, edge_index, edge_attr, params):
    A, Q, P3, P5, cent = _selectors()
    A = jnp.asarray(A)
    Q = jnp.asarray(Q)
    P3 = jnp.asarray(P3)
    P5 = jnp.asarray(P5)
    cent = jnp.asarray(cent)

    src = edge_index[0]
    dst = edge_index[1]
    pad = _EPAD - _E
    srcp = jnp.concatenate([src, jnp.zeros((pad,), jnp.int32)])
    dstp = jnp.concatenate([dst, jnp.full((pad,), _N, jnp.int32)])
    eap = jnp.concatenate(
        [edge_attr, jnp.ones((pad, 3), jnp.float32)], axis=0)
    idx2d = srcp.reshape(1, _EPAD)
    dst3 = dstp.reshape(_NROWS, 128)

    feat = x
    for i in range(_LAYERS):
        g = _sc_gather(feat, idx2d)
        msg = _tc_edge_msgs(eap, g, params, str(i), A, Q, cent)
        o = _sc_scatter(msg, dst3)
        feat = _tc_act(o, P3, P5)

    gf = _sc_gather(feat, idx2d)
    msgf = _tc_final_msgs(eap, gf, params, cent)
    return _sc_scatter(msgf, dst3)


# speedup vs baseline: 1.5106x; 1.5106x over previous
import functools
import numpy as np
import jax
import jax.numpy as jnp
from jax import lax
from jax.experimental import pallas as pl
from jax.experimental.pallas import tpu as pltpu
from jax.experimental.pallas import tpu_sc as plsc

_N = 10000
_E = 160000
_MUL = 8
_NB = 10
_H = 100
_LAYERS = 3
_MSGC = 176
_MSGP = 192
_OUTC = 128
_INV_SQRT_NNORM = 0.25

_GW = 128
_EPAD = 163840
_NROWS = _EPAD // 128
_NWORK = 32
_RPW = _NROWS // _NWORK
_NBIN = 10016
_RSUB = _NBIN // 16
_WSUB = _N // 16

_BE = 1280
_BN = 1000

_PREC = lax.Precision.DEFAULT


def _selectors():
    A = np.zeros((80, _MSGP), np.float32)
    for i in range(48):
        A[i, i] = 1.0
    for k in range(16):
        for j in range(3):
            A[48 + k, 48 + 3 * k + j] = 1.0
        for j in range(5):
            A[64 + k, 96 + 5 * k + j] = 1.0
    Q = np.zeros((9, _MSGP), np.float32)
    Q[0, :48] = 1.0
    for k in range(16):
        for j in range(3):
            Q[1 + j, 48 + 3 * k + j] = 1.0
        for j in range(5):
            Q[4 + j, 96 + 5 * k + j] = 1.0
    P3 = np.zeros((32, 48), np.float32)
    P5 = np.zeros((32, 80), np.float32)
    for k in range(16):
        for j in range(3):
            P3[k, 3 * k + j] = 1.0
        for j in range(5):
            P5[16 + k, 5 * k + j] = 1.0
    cent = np.linspace(0.0, 1.0, _NB).reshape(1, _NB).astype(np.float32)
    return A, Q, P3, P5, cent


def _sc_gather(feat, idx2d):
    cin = feat.shape[1]
    n_steps = idx2d.shape[1] // _GW
    mesh = plsc.VectorSubcoreMesh(core_axis_name="c", subcore_axis_name="s")

    def body(feat_hbm, i_hbm, o_hbm):
        def inner(i_vmem, o_vmem):
            pltpu.sync_copy(feat_hbm.at[i_vmem.at[0]], o_vmem)

        pltpu.emit_pipeline(
            inner,
            grid=(n_steps,),
            in_specs=[pl.BlockSpec((1, _GW), lambda i: (0, i))],
            out_specs=[pl.BlockSpec((_GW, cin), lambda i: (i, 0))],
            core_axis_name=("c", "s"),
            dimension_semantics=(pltpu.PARALLEL,),
        )(i_hbm, o_hbm)

    f = pl.kernel(
        body,
        out_type=jax.ShapeDtypeStruct((idx2d.shape[1], cin), jnp.float32),
        mesh=mesh,
        compiler_params=pltpu.CompilerParams(use_tc_tiling_on_sc=False),
    )
    return f(feat, idx2d)


def _sc_scatter(msg, dst3):
    C = msg.shape[1]
    CW = C // 2
    RPS = _NROWS // 16

    mesh = plsc.VectorSubcoreMesh(core_axis_name="c", subcore_axis_name="s")

    def body(msg_hbm, dst_hbm, out_hbm, idx_v, blk_v, acc_sh):
        c = lax.axis_index("c")
        s = lax.axis_index("s")
        coff = c * CW

        @pl.loop(0, 128)
        def _(r):
            @pl.loop(0, CW // 16)
            def _(j):
                blk_v[r, pl.ds(j * 16, 16)] = jnp.zeros((16,), jnp.float32)

        zbase = s * _RSUB

        @pl.loop(0, 4)
        def _(k):
            pltpu.sync_copy(blk_v, acc_sh.at[pl.ds(zbase + k * 128, 128)])

        pltpu.sync_copy(
            blk_v.at[pl.ds(0, _RSUB - 512)],
            acc_sh.at[pl.ds(zbase + 512, _RSUB - 512)],
        )
        plsc.subcore_barrier()

        @pl.loop(0, RPS)
        def _(k):
            r = s * RPS + k
            pltpu.sync_copy(dst_hbm.at[r], idx_v)
            pltpu.sync_copy(
                msg_hbm.at[pl.ds(r * 128, 128), pl.ds(coff, CW)], blk_v)
            pltpu.sync_copy(blk_v, acc_sh.at[idx_v], add=True)

        plsc.subcore_barrier()

        wbase = s * _WSUB
        pltpu.sync_copy(
            acc_sh.at[pl.ds(wbase, _WSUB)],
            out_hbm.at[pl.ds(wbase, _WSUB), pl.ds(coff, CW)],
        )

    f = pl.kernel(
        body,
        out_type=jax.ShapeDtypeStruct((_N, C), jnp.float32),
        mesh=mesh,
        scratch_types=[
            pltpu.VMEM((128,), jnp.int32),
            pltpu.VMEM((128, CW), jnp.float32),
            pltpu.VMEM_SHARED((_NBIN, CW), jnp.float32),
        ],
        compiler_params=pltpu.CompilerParams(use_tc_tiling_on_sc=False),
    )
    return f(msg, dst3)


def _edge_body(ea_ref, g_ref, w1, b1, w2, b2, w3, b3, wmix, a_ref, q_ref,
               cent_ref, msg_ref):
    ea = ea_ref[...]
    r2 = jnp.sum(ea * ea, axis=1, keepdims=True) + 1e-12
    r = jnp.sqrt(r2)
    inv = 1.0 / r
    xx = ea[:, 0:1] * inv
    yy = ea[:, 1:2] * inv
    zz = ea[:, 2:3] * inv

    d = (r - cent_ref[...]) * np.float32(_NB)
    B = jnp.exp(-0.5 * d * d)

    h1 = jnp.dot(B, w1[...], precision=_PREC) + b1[...]
    h1 = h1 * jax.nn.sigmoid(h1)
    h2 = jnp.dot(h1, w2[...], precision=_PREC) + b2[...]
    h2 = h2 * jax.nn.sigmoid(h2)
    R = jnp.dot(h2, w3[...], precision=_PREC) + b3[...]

    s = g_ref[...] * R
    h = jnp.dot(s, wmix[...], precision=_PREC)
    hA = jnp.dot(h, a_ref[...], precision=_PREC)

    q = q_ref[...]
    sq3 = np.float32(np.sqrt(3.0))
    ymul = (q[0:1, :]
            + xx * q[1:2, :] + yy * q[2:3, :] + zz * q[3:4, :]
            + (sq3 * xx * yy) * q[4:5, :]
            + (sq3 * yy * zz) * q[5:6, :]
            + ((3.0 * zz * zz - 1.0) * 0.5) * q[6:7, :]
            + (sq3 * xx * zz) * q[7:8, :]
            + (sq3 * 0.5 * (xx * xx - yy * yy)) * q[8:9, :])

    msg_ref[...] = hA * ymul * np.float32(_INV_SQRT_NNORM)


def _tc_edge_msgs(eap, g, p, tag, A, Q, cent):
    cin = g.shape[1]
    grid = (_EPAD // _BE,)
    full = lambda shape: pl.BlockSpec(shape, lambda i: (0, 0))
    return pl.pallas_call(
        _edge_body,
        grid=grid,
        in_specs=[
            pl.BlockSpec((_BE, 3), lambda i: (i, 0)),
            pl.BlockSpec((_BE, cin), lambda i: (i, 0)),
            full((_NB, _H)),
            full((1, _H)),
            full((_H, _H)),
            full((1, _H)),
            full((_H, cin)),
            full((1, cin)),
            full((cin, 80)),
            full((80, _MSGP)),
            full((9, _MSGP)),
            full((1, _NB)),
        ],
        out_specs=pl.BlockSpec((_BE, _MSGP), lambda i: (i, 0)),
        out_shape=jax.ShapeDtypeStruct((_EPAD, _MSGP), jnp.float32),
    )(eap, g,
      p['W1_' + tag], p['b1_' + tag].reshape(1, _H),
      p['W2_' + tag], p['b2_' + tag].reshape(1, _H),
      p['W3_' + tag], p['b3_' + tag].reshape(1, cin),
      p['Wmix_' + tag], A, Q, cent)


def _final_body(ea_ref, g_ref, w1, b1, w2, b2, w3, b3, wout, cent_ref,
                msg_ref):
    ea = ea_ref[...]
    r2 = jnp.sum(ea * ea, axis=1, keepdims=True) + 1e-12
    r = jnp.sqrt(r2)
    d = (r - cent_ref[...]) * np.float32(_NB)
    B = jnp.exp(-0.5 * d * d)
    h1 = jnp.dot(B, w1[...], precision=_PREC) + b1[...]
    h1 = h1 * jax.nn.sigmoid(h1)
    h2 = jnp.dot(h1, w2[...], precision=_PREC) + b2[...]
    h2 = h2 * jax.nn.sigmoid(h2)
    R = jnp.dot(h2, w3[...], precision=_PREC) + b3[...]
    s = g_ref[...] * R
    msg_ref[...] = jnp.dot(s, wout[...], precision=_PREC) * np.float32(
        _INV_SQRT_NNORM)


def _tc_final_msgs(eap, g, p, cent):
    cin = g.shape[1]
    grid = (_EPAD // _BE,)
    full = lambda shape: pl.BlockSpec(shape, lambda i: (0, 0))
    return pl.pallas_call(
        _final_body,
        grid=grid,
        in_specs=[
            pl.BlockSpec((_BE, 3), lambda i: (i, 0)),
            pl.BlockSpec((_BE, cin), lambda i: (i, 0)),
            full((_NB, _H)),
            full((1, _H)),
            full((_H, _H)),
            full((1, _H)),
            full((_H, cin)),
            full((1, cin)),
            full((cin, _OUTC)),
            full((1, _NB)),
        ],
        out_specs=pl.BlockSpec((_BE, _OUTC), lambda i: (i, 0)),
        out_shape=jax.ShapeDtypeStruct((_EPAD, _OUTC), jnp.float32),
    )(eap, g,
      p['W1_f'], p['b1_f'].reshape(1, _H),
      p['W2_f'], p['b2_f'].reshape(1, _H),
      p['W3_f'], p['b3_f'].reshape(1, cin),
      p['Wout'], cent)


def _act_body(o_ref, p3_ref, p5_ref, feat_ref):
    o = o_ref[...]
    se = o[:, :8]
    se = se * jax.nn.sigmoid(se)
    so = jnp.tanh(o[:, 8:16])
    g = jax.nn.sigmoid(o[:, 16:48])
    v1 = o[:, 48:96] * jnp.dot(g, p3_ref[...], precision=_PREC)
    v2 = o[:, 96:176] * jnp.dot(g, p5_ref[...], precision=_PREC)
    feat_ref[...] = jnp.concatenate([se, so, v1, v2], axis=1)


def _tc_act(o, P3, P5):
    grid = (_N // _BN,)
    full = lambda shape: pl.BlockSpec(shape, lambda i: (0, 0))
    return pl.pallas_call(
        _act_body,
        grid=grid,
        in_specs=[
            pl.BlockSpec((_BN, _MSGP), lambda i: (i, 0)),
            full((32, 48)),
            full((32, 80)),
        ],
        out_specs=pl.BlockSpec((_BN, 144), lambda i: (i, 0)),
        out_shape=jax.ShapeDtypeStruct((_N, 144), jnp.float32),
    )(o, P3, P5)


def kernel(x, edge_index, edge_attr, params):
    A, Q, P3, P5, cent = _selectors()
    A = jnp.asarray(A)
    Q = jnp.asarray(Q)
    P3 = jnp.asarray(P3)
    P5 = jnp.asarray(P5)
    cent = jnp.asarray(cent)

    src = edge_index[0]
    dst = edge_index[1]
    pad = _EPAD - _E
    srcp = jnp.concatenate([src, jnp.zeros((pad,), jnp.int32)])
    dstp = jnp.concatenate([dst, jnp.full((pad,), _N, jnp.int32)])
    eap = jnp.concatenate(
        [edge_attr, jnp.ones((pad, 3), jnp.float32)], axis=0)
    idx2d = srcp.reshape(1, _EPAD)
    dst3 = dstp.reshape(_NROWS, 128)

    feat = x
    for i in range(_LAYERS):
        g = _sc_gather(feat, idx2d)
        msg = _tc_edge_msgs(eap, g, params, str(i), A, Q, cent)
        o = _sc_scatter(msg, dst3)
        feat = _tc_act(o, P3, P5)

    gf = _sc_gather(feat, idx2d)
    msgf = _tc_final_msgs(eap, gf, params, cent)
    return _sc_scatter(msgf, dst3)
